# baseline (speedup 1.0000x reference)
import functools

import jax
import jax.numpy as jnp
from jax import lax
from jax.experimental import pallas as pl
from jax.experimental.pallas import tpu as pltpu
from jax.experimental.pallas import tpu_sc as plsc

NUM_ROWS = 1000000
DIM = 32
LANES = NUM_ROWS
CHUNK = 1536
FULL = 7812 * 128
N_CHUNKS = FULL // CHUNK
PER_W = 21
TAIL = LANES - FULL


def _sc_copy_body(t_hbm, out_hbm, buf0, buf1, tailbuf, is0, is1, os0, os1, ts):
    wid = lax.axis_index("s") * 2 + lax.axis_index("c")
    start = (wid * (N_CHUNKS - PER_W)) // 31
    bufs = (buf0, buf1)
    isems = (is0, is1)
    osems = (os0, os1)

    def lane0(k):
        return pl.multiple_of((start + k) * CHUNK, 128)

    pltpu.async_copy(t_hbm.at[:, pl.ds(lane0(0), CHUNK)], buf0, is0)
    for k in range(PER_W):
        b = k % 2
        pltpu.make_async_copy(
            t_hbm.at[:, pl.ds(lane0(k), CHUNK)], bufs[b], isems[b]
        ).wait()
        if k >= 1:
            pltpu.make_async_copy(
                bufs[1 - b],
                out_hbm.at[:, pl.ds(lane0(k - 1), CHUNK)],
                osems[1 - b],
            ).wait()
        if k + 1 < PER_W:
            pltpu.async_copy(
                t_hbm.at[:, pl.ds(lane0(k + 1), CHUNK)], bufs[1 - b], isems[1 - b]
            )
        pltpu.async_copy(
            bufs[b], out_hbm.at[:, pl.ds(lane0(k), CHUNK)], osems[b]
        )

    @pl.when(wid == 0)
    def _():
        pltpu.make_async_copy(t_hbm.at[:, pl.ds(FULL, TAIL)], tailbuf, ts).start()
        pltpu.make_async_copy(t_hbm.at[:, pl.ds(FULL, TAIL)], tailbuf, ts).wait()
        pltpu.make_async_copy(tailbuf, out_hbm.at[:, pl.ds(FULL, TAIL)], ts).start()
        pltpu.make_async_copy(tailbuf, out_hbm.at[:, pl.ds(FULL, TAIL)], ts).wait()

    lastb = (PER_W - 1) % 2
    pltpu.make_async_copy(
        bufs[lastb],
        out_hbm.at[:, pl.ds(lane0(PER_W - 1), CHUNK)],
        osems[lastb],
    ).wait()


def kernel(table):
    sc_copy = pl.kernel(
        _sc_copy_body,
        mesh=plsc.VectorSubcoreMesh(core_axis_name="c", subcore_axis_name="s"),
        out_type=jax.ShapeDtypeStruct((DIM, LANES), jnp.float32),
        scratch_types=[
            pltpu.VMEM((DIM, CHUNK), jnp.float32),
            pltpu.VMEM((DIM, CHUNK), jnp.float32),
            pltpu.VMEM((DIM, TAIL), jnp.float32),
            pltpu.SemaphoreType.DMA,
            pltpu.SemaphoreType.DMA,
            pltpu.SemaphoreType.DMA,
            pltpu.SemaphoreType.DMA,
            pltpu.SemaphoreType.DMA,
        ],
    )
    return sc_copy(table.T).T


# speedup vs baseline: 1.0002x; 1.0002x over previous
import functools

import jax
import jax.numpy as jnp
from jax import lax
from jax.experimental import pallas as pl
from jax.experimental.pallas import tpu as pltpu
from jax.experimental.pallas import tpu_sc as plsc

NUM_ROWS = 1000000
DIM = 32
LANES = NUM_ROWS
CHUNK = 1536
FULL = 7812 * 128
N_CHUNKS = FULL // CHUNK
PER_W = 21
TAIL = LANES - FULL


def _sc_copy_body(t_hbm, out_hbm, buf0, buf1, tailbuf, is0, is1, os0, os1, ts):
    wid = lax.axis_index("s") * 2 + lax.axis_index("c")
    start = (wid * (N_CHUNKS - PER_W)) // 31
    bufs = (buf0, buf1)
    isems = (is0, is1)
    osems = (os0, os1)

    def lane0(k):
        return pl.multiple_of((start + k) * CHUNK, 128)

    pltpu.async_copy(t_hbm.at[:, pl.ds(lane0(0), CHUNK)], buf0, is0)
    for k in range(PER_W):
        b = k % 2
        pltpu.make_async_copy(
            t_hbm.at[:, pl.ds(lane0(k), CHUNK)], bufs[b], isems[b]
        ).wait()
        if k >= 1:
            pltpu.make_async_copy(
                bufs[1 - b],
                out_hbm.at[:, pl.ds(lane0(k - 1), CHUNK)],
                osems[1 - b],
            ).wait()
        if k + 1 < PER_W:
            pltpu.async_copy(
                t_hbm.at[:, pl.ds(lane0(k + 1), CHUNK)], bufs[1 - b], isems[1 - b]
            )
        pltpu.async_copy(
            bufs[b], out_hbm.at[:, pl.ds(lane0(k), CHUNK)], osems[b]
        )

    @pl.when(wid == 0)
    def _():
        pltpu.make_async_copy(t_hbm.at[:, pl.ds(FULL, TAIL)], tailbuf, ts).start()
        pltpu.make_async_copy(t_hbm.at[:, pl.ds(FULL, TAIL)], tailbuf, ts).wait()
        pltpu.make_async_copy(tailbuf, out_hbm.at[:, pl.ds(FULL, TAIL)], ts).start()
        pltpu.make_async_copy(tailbuf, out_hbm.at[:, pl.ds(FULL, TAIL)], ts).wait()

    lastb = (PER_W - 1) % 2
    pltpu.make_async_copy(
        bufs[lastb],
        out_hbm.at[:, pl.ds(lane0(PER_W - 1), CHUNK)],
        osems[lastb],
    ).wait()


def kernel(table):
    sc_copy = pl.kernel(
        _sc_copy_body,
        mesh=plsc.VectorSubcoreMesh(core_axis_name="c", subcore_axis_name="s"),
        compiler_params=pltpu.CompilerParams(skip_device_barrier=True),
        out_type=jax.ShapeDtypeStruct((DIM, LANES), jnp.float32),
        scratch_types=[
            pltpu.VMEM((DIM, CHUNK), jnp.float32),
            pltpu.VMEM((DIM, CHUNK), jnp.float32),
            pltpu.VMEM((DIM, TAIL), jnp.float32),
            pltpu.SemaphoreType.DMA,
            pltpu.SemaphoreType.DMA,
            pltpu.SemaphoreType.DMA,
            pltpu.SemaphoreType.DMA,
            pltpu.SemaphoreType.DMA,
        ],
    )
    return sc_copy(table.T).T


# speedup vs baseline: 1.0041x; 1.0039x over previous
import functools

import jax
import jax.numpy as jnp
from jax import lax
from jax.experimental import pallas as pl
from jax.experimental.pallas import tpu as pltpu
from jax.experimental.pallas import tpu_sc as plsc

NUM_ROWS = 1000000
DIM = 32
LANES = NUM_ROWS
CHUNK = 1536
FULL = 7812 * 128
N_CHUNKS = FULL // CHUNK
PER_W = 21
TAIL = LANES - FULL


def _sc_copy_body(t_hbm, out_hbm, buf0, buf1, tailbuf, is0, is1, os0, os1, ts):
    wid = lax.axis_index("s") * 2 + lax.axis_index("c")
    start = (wid * (N_CHUNKS - PER_W)) // 31
    bufs = (buf0, buf1)
    isems = (is0, is1)
    osems = (os0, os1)

    def lane0(k):
        return pl.multiple_of((start + k) * CHUNK, 128)

    pltpu.async_copy(t_hbm.at[:, pl.ds(lane0(0), CHUNK)], buf0, is0)
    for k in range(PER_W):
        b = k % 2
        pltpu.make_async_copy(
            t_hbm.at[:, pl.ds(lane0(k), CHUNK)], bufs[b], isems[b]
        ).wait()
        if k >= 1:
            pltpu.make_async_copy(
                bufs[1 - b],
                out_hbm.at[:, pl.ds(lane0(k - 1), CHUNK)],
                osems[1 - b],
            ).wait()
        if k + 1 < PER_W:
            pltpu.async_copy(
                t_hbm.at[:, pl.ds(lane0(k + 1), CHUNK)], bufs[1 - b], isems[1 - b]
            )
        pltpu.async_copy(
            bufs[b], out_hbm.at[:, pl.ds(lane0(k), CHUNK)], osems[b]
        )

    @pl.when(wid == 0)
    def _():
        pltpu.make_async_copy(t_hbm.at[:, pl.ds(FULL, TAIL)], tailbuf, ts).start()
        pltpu.make_async_copy(t_hbm.at[:, pl.ds(FULL, TAIL)], tailbuf, ts).wait()
        pltpu.make_async_copy(tailbuf, out_hbm.at[:, pl.ds(FULL, TAIL)], ts).start()
        pltpu.make_async_copy(tailbuf, out_hbm.at[:, pl.ds(FULL, TAIL)], ts).wait()

    lastb = (PER_W - 1) % 2
    pltpu.make_async_copy(
        bufs[lastb],
        out_hbm.at[:, pl.ds(lane0(PER_W - 1), CHUNK)],
        osems[lastb],
    ).wait()


def kernel(table):
    sc_copy = pl.kernel(
        _sc_copy_body,
        mesh=plsc.VectorSubcoreMesh(core_axis_name="c", subcore_axis_name="s"),
        out_type=jax.ShapeDtypeStruct((DIM, LANES), jnp.float32),
        scratch_types=[
            pltpu.VMEM((DIM, CHUNK), jnp.float32),
            pltpu.VMEM((DIM, CHUNK), jnp.float32),
            pltpu.VMEM((DIM, TAIL), jnp.float32),
            pltpu.SemaphoreType.DMA,
            pltpu.SemaphoreType.DMA,
            pltpu.SemaphoreType.DMA,
            pltpu.SemaphoreType.DMA,
            pltpu.SemaphoreType.DMA,
        ],
    )
    return sc_copy(table.T).T
